# Initial kernel scaffold; baseline (speedup 1.0000x reference)
import jax
import jax.numpy as jnp
from jax.experimental import pallas as pl


def kernel(x, table):
    raise NotImplementedError("write your pallas kernel here")


# speedup vs baseline: 1.6861x; 1.6861x over previous
import functools

import jax
import jax.numpy as jnp
from jax import lax
from jax.experimental import pallas as pl
from jax.experimental.pallas import tpu as pltpu
from jax.experimental.pallas import tpu_sc as plsc

BATCH = 16384
HIST = 50
EMBED_DIM = 64

NUM_CORES = 2
NUM_SUBCORES = 16
NW = NUM_CORES * NUM_SUBCORES
CHUNK = 128
B_TOTAL = BATCH * HIST
STEPS = B_TOTAL // (NW * CHUNK)

_mesh = plsc.VectorSubcoreMesh(core_axis_name="c", subcore_axis_name="s")


@functools.partial(
    pl.kernel,
    mesh=_mesh,
    compiler_params=pltpu.CompilerParams(use_tc_tiling_on_sc=False),
    out_type=jax.ShapeDtypeStruct((NW, STEPS, CHUNK, EMBED_DIM), jnp.float32),
    scratch_types=[
        pltpu.VMEM((STEPS, CHUNK), jnp.int32),
        pltpu.VMEM((CHUNK, EMBED_DIM), jnp.float32),
        pltpu.SemaphoreType.DMA,
    ],
)
def _gather_kernel(table_hbm, idx_hbm, out_hbm, idx_v, rows_v, sem):
    wid = lax.axis_index("s") * NUM_CORES + lax.axis_index("c")
    pltpu.sync_copy(idx_hbm.at[wid], idx_v)

    def body(j, _):
        pltpu.async_copy(table_hbm.at[idx_v.at[j]], rows_v, sem).wait()
        pltpu.sync_copy(rows_v, out_hbm.at[wid].at[j])
        return ()

    lax.fori_loop(0, STEPS, body, ())


def kernel(x, table):
    xf = x.reshape(NW, STEPS, CHUNK).astype(jnp.int32)
    out = _gather_kernel(table, xf)
    return out.reshape(BATCH, HIST, EMBED_DIM)


# speedup vs baseline: 1.8691x; 1.1085x over previous
import functools

import jax
import jax.numpy as jnp
from jax import lax
from jax.experimental import pallas as pl
from jax.experimental.pallas import tpu as pltpu
from jax.experimental.pallas import tpu_sc as plsc

BATCH = 16384
HIST = 50
EMBED_DIM = 64

NUM_CORES = 2
NUM_SUBCORES = 16
NW = NUM_CORES * NUM_SUBCORES
CHUNK = 128
B_TOTAL = BATCH * HIST
STEPS = B_TOTAL // (NW * CHUNK)
GPB = 4
BANK_ROWS = GPB * CHUNK
NSUPER = STEPS // GPB

_mesh = plsc.VectorSubcoreMesh(core_axis_name="c", subcore_axis_name="s")


@functools.partial(
    pl.kernel,
    mesh=_mesh,
    compiler_params=pltpu.CompilerParams(use_tc_tiling_on_sc=False),
    out_type=jax.ShapeDtypeStruct((NW, NSUPER, BANK_ROWS, EMBED_DIM), jnp.float32),
    scratch_types=[
        pltpu.VMEM((STEPS, CHUNK), jnp.int32),
        pltpu.VMEM((2, BANK_ROWS, EMBED_DIM), jnp.float32),
        pltpu.SemaphoreType.DMA,
        pltpu.SemaphoreType.DMA,
    ],
)
def _gather_kernel(table_hbm, idx_hbm, out_hbm, idx_v, rows_v, gsem, wsem):
    wid = lax.axis_index("s") * NUM_CORES + lax.axis_index("c")
    pltpu.sync_copy(idx_hbm.at[wid], idx_v)

    def start_bank(t, bank):
        for u in range(GPB):
            pltpu.async_copy(
                table_hbm.at[idx_v.at[GPB * t + u]],
                rows_v.at[bank, pl.ds(u * CHUNK, CHUNK)],
                gsem,
            )

    def wait_bank(bank):
        pltpu.make_async_copy(
            table_hbm.at[idx_v.at[0]], rows_v.at[bank], gsem
        ).wait()

    start_bank(0, 0)

    def step(t, _):
        cur = t % 2
        wait_bank(cur)

        @pl.when(t > 0)
        def _():
            pltpu.make_async_copy(
                rows_v.at[1 - cur], out_hbm.at[wid, 0], wsem
            ).wait()

        pltpu.async_copy(rows_v.at[cur], out_hbm.at[wid, t], wsem)

        @pl.when(t < NSUPER - 1)
        def _():
            start_bank(t + 1, 1 - cur)

        return ()

    lax.fori_loop(0, NSUPER, step, ())
    pltpu.make_async_copy(
        rows_v.at[(NSUPER - 1) % 2], out_hbm.at[wid, 0], wsem
    ).wait()


def kernel(x, table):
    xf = x.reshape(NW, STEPS, CHUNK).astype(jnp.int32)
    out = _gather_kernel(table, xf)
    return out.reshape(BATCH, HIST, EMBED_DIM)
